# Initial kernel scaffold; baseline (speedup 1.0000x reference)
import functools

import jax
import jax.numpy as jnp
from jax import lax
from jax.experimental import pallas as pl
from jax.experimental.pallas import tpu as pltpu
from jax.experimental.pallas import tpu_sc as plsc

VOCAB = 1000000
D = 64
B = 16384
NEG = 10

_info = plsc.get_sparse_core_info()
NC, NS, L = _info.num_cores, _info.num_subcores, _info.num_lanes
NW = NC * NS
BPW = B // NW
C = 64
ROUNDS = BPW // C
KOUT = 1 + NEG


def _sc_scores(pos_u, pos_v, neg_flat, u_table, v_table):
    mesh = plsc.VectorSubcoreMesh(core_axis_name="c", subcore_axis_name="s")

    @functools.partial(
        pl.kernel,
        mesh=mesh,
        out_type=jax.ShapeDtypeStruct((NW, KOUT, BPW), jnp.float32),
        scratch_types=[
            pltpu.VMEM((BPW,), jnp.int32),
            pltpu.VMEM((BPW,), jnp.int32),
            pltpu.VMEM((BPW * NEG,), jnp.int32),
            pltpu.VMEM((C, D), jnp.float32),
            pltpu.VMEM((C, D), jnp.float32),
            pltpu.VMEM((C * NEG, D), jnp.float32),
            pltpu.VMEM((KOUT, BPW), jnp.float32),
            pltpu.SemaphoreType.DMA,
            pltpu.SemaphoreType.DMA,
            pltpu.SemaphoreType.DMA,
        ],
    )
    def k(pos_u_h, pos_v_h, neg_h, u_tab, v_tab, out_h,
          u_idx, v_idx, n_idx, u_rows, v_rows, n_rows, stage,
          usem, vsem, nsem):
        wid = lax.axis_index("s") * NC + lax.axis_index("c")
        base = wid * BPW
        pltpu.sync_copy(pos_u_h.at[pl.ds(base, BPW)], u_idx)
        pltpu.sync_copy(pos_v_h.at[pl.ds(base, BPW)], v_idx)
        pltpu.sync_copy(neg_h.at[pl.ds(base * NEG, BPW * NEG)], n_idx)

        for r in range(ROUNDS):
            cu = pltpu.async_copy(u_tab.at[u_idx.at[pl.ds(r * C, C)]], u_rows, usem)
            cv = pltpu.async_copy(v_tab.at[v_idx.at[pl.ds(r * C, C)]], v_rows, vsem)
            cn = pltpu.async_copy(
                v_tab.at[n_idx.at[pl.ds(r * C * NEG, C * NEG)]], n_rows, nsem)
            cu.wait()
            cv.wait()
            cn.wait()

            for g in range(C // L):
                e = lax.iota(jnp.int32, L) + (g * L)
                e10 = e * NEG

                def dbody(d, accs, e=e, e10=e10):
                    dv = jnp.full((L,), d, jnp.int32)
                    u_d = plsc.load_gather(u_rows, [e, dv])
                    v_d = plsc.load_gather(v_rows, [e, dv])
                    pos = accs[0] + u_d * v_d
                    negs = [
                        accs[1 + kk] - plsc.load_gather(n_rows, [e10 + kk, dv]) * u_d
                        for kk in range(NEG)
                    ]
                    return (pos, *negs)

                accs = lax.fori_loop(
                    0, D, dbody,
                    tuple(jnp.zeros((L,), jnp.float32) for _ in range(KOUT)))
                col = r * C + g * L
                for kk in range(KOUT):
                    stage[kk, pl.ds(col, L)] = accs[kk]

        pltpu.sync_copy(stage, out_h.at[wid])

    return k(pos_u, pos_v, neg_flat, u_table, v_table)


def _tc_loss(scores2d):
    def body(s_ref, o_ref):
        x = s_ref[...]
        ls = jnp.where(x < 0.0, x, 0.0) - jnp.log1p(jnp.exp(-jnp.abs(x)))
        o_ref[0, 0] = -jnp.sum(ls)

    return pl.pallas_call(
        body,
        out_shape=jax.ShapeDtypeStruct((1, 1), jnp.float32),
        out_specs=pl.BlockSpec(memory_space=pltpu.SMEM),
    )(scores2d)


@jax.jit
def kernel(pos_u, pos_v, neg_v, u_table, v_table):
    neg_flat = neg_v.astype(jnp.int32).reshape(-1)
    scores = _sc_scores(pos_u.astype(jnp.int32), pos_v.astype(jnp.int32),
                        neg_flat, u_table, v_table)
    loss = _tc_loss(scores.reshape(NW * KOUT, BPW))
    return loss[0, 0]


# speedup vs baseline: 2.5362x; 2.5362x over previous
import functools

import jax
import jax.numpy as jnp
from jax import lax
from jax.experimental import pallas as pl
from jax.experimental.pallas import tpu as pltpu
from jax.experimental.pallas import tpu_sc as plsc

VOCAB = 1000000
D = 64
B = 16384
NEG = 10

NC, NS, L = 2, 16, 16
NW = NC * NS
BPW = B // NW
C = 64
ROUNDS = BPW // C
KOUT = 1 + NEG


def _sc_scores(pos_u, pos_v, neg_flat, u_table, v_table):
    mesh = plsc.VectorSubcoreMesh(
        core_axis_name="c", subcore_axis_name="s",
        num_cores=NC, num_subcores=NS)

    @functools.partial(
        pl.kernel,
        mesh=mesh,
        compiler_params=pltpu.CompilerParams(
            needs_layout_passes=False, use_tc_tiling_on_sc=False),
        out_type=jax.ShapeDtypeStruct((NW, KOUT, BPW), jnp.float32),
        scratch_types=[
            pltpu.VMEM((BPW,), jnp.int32),
            pltpu.VMEM((BPW,), jnp.int32),
            pltpu.VMEM((BPW * NEG,), jnp.int32),
            pltpu.VMEM((C, D), jnp.float32),
            pltpu.VMEM((C, D), jnp.float32),
            pltpu.VMEM((C * NEG, D), jnp.float32),
            pltpu.VMEM((KOUT, BPW), jnp.float32),
            pltpu.SemaphoreType.DMA,
            pltpu.SemaphoreType.DMA,
            pltpu.SemaphoreType.DMA,
        ],
    )
    def k(pos_u_h, pos_v_h, neg_h, u_tab, v_tab, out_h,
          u_idx, v_idx, n_idx, u_rows, v_rows, n_rows, stage,
          usem, vsem, nsem):
        wid = lax.axis_index("s") * NC + lax.axis_index("c")
        base = wid * BPW
        pltpu.sync_copy(pos_u_h.at[pl.ds(base, BPW)], u_idx)
        pltpu.sync_copy(pos_v_h.at[pl.ds(base, BPW)], v_idx)
        pltpu.sync_copy(neg_h.at[pl.ds(base * NEG, BPW * NEG)], n_idx)

        for r in range(ROUNDS):
            cu = pltpu.async_copy(u_tab.at[u_idx.at[pl.ds(r * C, C)]], u_rows, usem)
            cv = pltpu.async_copy(v_tab.at[v_idx.at[pl.ds(r * C, C)]], v_rows, vsem)
            cn = pltpu.async_copy(
                v_tab.at[n_idx.at[pl.ds(r * C * NEG, C * NEG)]], n_rows, nsem)
            cu.wait()
            cv.wait()
            cn.wait()

            for g in range(C // L):
                e = lax.iota(jnp.int32, L) + (g * L)
                e10 = e * NEG

                def dbody(d, accs, e=e, e10=e10):
                    dv = jnp.full((L,), d, jnp.int32)
                    u_d = plsc.load_gather(u_rows, [e, dv])
                    v_d = plsc.load_gather(v_rows, [e, dv])
                    pos = accs[0] + u_d * v_d
                    negs = [
                        accs[1 + kk] - plsc.load_gather(n_rows, [e10 + kk, dv]) * u_d
                        for kk in range(NEG)
                    ]
                    return (pos, *negs)

                accs = lax.fori_loop(
                    0, D, dbody,
                    tuple(jnp.zeros((L,), jnp.float32) for _ in range(KOUT)))
                col = r * C + g * L
                for kk in range(KOUT):
                    stage[kk, pl.ds(col, L)] = accs[kk]

        pltpu.sync_copy(stage, out_h.at[wid])

    return k(pos_u, pos_v, neg_flat, u_table, v_table)


def _tc_loss(scores2d):
    def body(s_ref, o_ref):
        x = s_ref[...]
        ls = jnp.where(x < 0.0, x, 0.0) - jnp.log1p(jnp.exp(-jnp.abs(x)))
        o_ref[0, 0] = -jnp.sum(ls)

    return pl.pallas_call(
        body,
        out_shape=jax.ShapeDtypeStruct((1, 1), jnp.float32),
        out_specs=pl.BlockSpec(memory_space=pltpu.SMEM),
    )(scores2d)


@jax.jit
def kernel(pos_u, pos_v, neg_v, u_table, v_table):
    neg_flat = neg_v.astype(jnp.int32).reshape(-1)
    scores = _sc_scores(pos_u.astype(jnp.int32), pos_v.astype(jnp.int32),
                        neg_flat, u_table, v_table)
    loss = _tc_loss(scores.reshape(NW * KOUT, BPW))
    return loss[0, 0]


# speedup vs baseline: 2.6233x; 1.0343x over previous
import functools

import jax
import jax.numpy as jnp
from jax import lax
from jax.experimental import pallas as pl
from jax.experimental.pallas import tpu as pltpu
from jax.experimental.pallas import tpu_sc as plsc

VOCAB = 1000000
D = 64
B = 16384
NEG = 10

NC, NS, L = 2, 16, 16
NW = NC * NS
BPW = B // NW
C = 32
ROUNDS = BPW // C
KOUT = 1 + NEG
PD = 2 * D


def _sc_scores(pos_u, pos_v, neg_flat, u2, v2):
    mesh = plsc.VectorSubcoreMesh(
        core_axis_name="c", subcore_axis_name="s",
        num_cores=NC, num_subcores=NS)

    @functools.partial(
        pl.kernel,
        mesh=mesh,
        compiler_params=pltpu.CompilerParams(
            needs_layout_passes=False, use_tc_tiling_on_sc=True),
        out_type=jax.ShapeDtypeStruct((NW, KOUT, BPW), jnp.float32),
        scratch_types=[
            pltpu.VMEM((BPW,), jnp.int32),
            pltpu.VMEM((BPW,), jnp.int32),
            pltpu.VMEM((BPW * NEG,), jnp.int32),
            pltpu.VMEM((BPW,), jnp.int32),
            pltpu.VMEM((BPW,), jnp.int32),
            pltpu.VMEM((BPW * NEG,), jnp.int32),
            pltpu.VMEM((2, C, PD), jnp.float32),
            pltpu.VMEM((2, C, PD), jnp.float32),
            pltpu.VMEM((2, C * NEG, PD), jnp.float32),
            pltpu.VMEM((KOUT, BPW), jnp.float32),
            pltpu.SemaphoreType.DMA,
            pltpu.SemaphoreType.DMA,
        ],
    )
    def k(pos_u_h, pos_v_h, neg_h, u_tab, v_tab, out_h,
          u_idx, v_idx, n_idx, u_phys, v_phys, n_phys,
          u_rows, v_rows, n_rows, stage, sem0, sem1):
        wid = lax.axis_index("s") * NC + lax.axis_index("c")
        base = wid * BPW
        pltpu.sync_copy(pos_u_h.at[pl.ds(base, BPW)], u_idx)
        pltpu.sync_copy(pos_v_h.at[pl.ds(base, BPW)], v_idx)
        pltpu.sync_copy(neg_h.at[pl.ds(base * NEG, BPW * NEG)], n_idx)

        def make_halver(src, dst):
            def halver(i, acc):
                dst[pl.ds(i * L, L)] = lax.shift_right_logical(
                    src[pl.ds(i * L, L)], 1)
                return acc
            return halver

        lax.fori_loop(0, BPW // L, make_halver(u_idx, u_phys), 0)
        lax.fori_loop(0, BPW // L, make_halver(v_idx, v_phys), 0)
        lax.fori_loop(0, BPW * NEG // L, make_halver(n_idx, n_phys), 0)

        sems = (sem0, sem1)

        NCHUNK = 128

        def fire(r, slot):
            cps = [
                pltpu.async_copy(
                    u_tab.at[u_phys.at[pl.ds(r * C, C)]], u_rows.at[slot],
                    sems[slot]),
                pltpu.async_copy(
                    v_tab.at[v_phys.at[pl.ds(r * C, C)]], v_rows.at[slot],
                    sems[slot]),
            ]
            for s in range(0, C * NEG, NCHUNK):
                n = min(NCHUNK, C * NEG - s)
                cps.append(pltpu.async_copy(
                    v_tab.at[n_phys.at[pl.ds(r * C * NEG + s, n)]],
                    n_rows.at[slot].at[pl.ds(s, n)], sems[slot]))
            return tuple(cps)

        UNROLL = 4
        iota = lax.iota(jnp.int32, L)
        pending = fire(0, 0)
        for r in range(ROUNDS):
            slot = r % 2
            for cpy in pending:
                cpy.wait()
            if r + 1 < ROUNDS:
                pending = fire(r + 1, 1 - slot)
            ur = u_rows.at[slot]
            vr = v_rows.at[slot]
            nr = n_rows.at[slot]

            def gbody(g, _, ur=ur, vr=vr, nr=nr, r=r):
                e = iota + g * L
                e10 = e * NEG
                col = r * C + g * L
                uo = lax.shift_left(
                    jnp.bitwise_and(u_idx[pl.ds(col, L)], 1), 6)
                vo = lax.shift_left(
                    jnp.bitwise_and(v_idx[pl.ds(col, L)], 1), 6)
                gpos10 = (iota + col) * NEG
                nos = [
                    lax.shift_left(
                        jnp.bitwise_and(
                            plsc.load_gather(n_idx, [gpos10 + kk]), 1), 6)
                    for kk in range(NEG)
                ]

                def dbody(j, accs, e=e, e10=e10, uo=uo, vo=vo, nos=nos,
                          ur=ur, vr=vr, nr=nr):
                    accs = list(accs)
                    for jj in range(UNROLL):
                        d = j * UNROLL + jj
                        u_d = plsc.load_gather(ur, [e, uo + d])
                        v_d = plsc.load_gather(vr, [e, vo + d])
                        accs[0] = accs[0] + u_d * v_d
                        for kk in range(NEG):
                            n_d = plsc.load_gather(nr, [e10 + kk, nos[kk] + d])
                            accs[1 + kk] = accs[1 + kk] - n_d * u_d
                    return tuple(accs)

                accs = lax.fori_loop(
                    0, D // UNROLL, dbody,
                    tuple(jnp.zeros((L,), jnp.float32) for _ in range(KOUT)))
                for kk in range(KOUT):
                    stage[kk, pl.ds(col, L)] = accs[kk]
                return 0

            lax.fori_loop(0, C // L, gbody, 0)

        pltpu.sync_copy(stage, out_h.at[wid])

    return k(pos_u, pos_v, neg_flat, u2, v2)


def _tc_loss(scores2d):
    def body(s_ref, o_ref):
        x = s_ref[...]
        ls = jnp.where(x < 0.0, x, 0.0) - jnp.log1p(jnp.exp(-jnp.abs(x)))
        o_ref[0, 0] = -jnp.sum(ls)

    return pl.pallas_call(
        body,
        out_shape=jax.ShapeDtypeStruct((1, 1), jnp.float32),
        out_specs=pl.BlockSpec(memory_space=pltpu.SMEM),
    )(scores2d)


@jax.jit
def kernel(pos_u, pos_v, neg_v, u_table, v_table):
    neg_flat = neg_v.astype(jnp.int32).reshape(-1)
    u2 = u_table.reshape(VOCAB // 2, PD)
    v2 = v_table.reshape(VOCAB // 2, PD)
    scores = _sc_scores(pos_u.astype(jnp.int32), pos_v.astype(jnp.int32),
                        neg_flat, u2, v2)
    loss = _tc_loss(scores.reshape(NW * KOUT, BPW))
    return loss[0, 0]


# speedup vs baseline: 3.0080x; 1.1467x over previous
import functools

import jax
import jax.numpy as jnp
from jax import lax
from jax.experimental import pallas as pl
from jax.experimental.pallas import tpu as pltpu
from jax.experimental.pallas import tpu_sc as plsc

VOCAB = 1000000
D = 64
B = 16384
NEG = 10

NC, NS, L = 2, 16, 16
NW = NC * NS
BPW = B // NW
C = 64
ROUNDS = BPW // C
KOUT = 1 + NEG


def _sc_scores(pos_u, pos_v, neg_flat, u_table, v_table):
    mesh = plsc.VectorSubcoreMesh(
        core_axis_name="c", subcore_axis_name="s",
        num_cores=NC, num_subcores=NS)

    @functools.partial(
        pl.kernel,
        mesh=mesh,
        compiler_params=pltpu.CompilerParams(
            needs_layout_passes=False, use_tc_tiling_on_sc=False),
        out_type=jax.ShapeDtypeStruct((NW, KOUT, BPW), jnp.float32),
        scratch_types=[
            pltpu.VMEM((BPW,), jnp.int32),
            pltpu.VMEM((BPW,), jnp.int32),
            pltpu.VMEM((BPW * NEG,), jnp.int32),
            pltpu.VMEM((2, C, D), jnp.float32),
            pltpu.VMEM((2, C, D), jnp.float32),
            pltpu.VMEM((2, C * NEG, D), jnp.float32),
            pltpu.VMEM((KOUT, BPW), jnp.float32),
            pltpu.SemaphoreType.DMA,
            pltpu.SemaphoreType.DMA,
        ],
    )
    def k(pos_u_h, pos_v_h, neg_h, u_tab, v_tab, out_h,
          u_idx, v_idx, n_idx, u_rows, v_rows, n_rows, stage, sem0, sem1):
        wid = lax.axis_index("s") * NC + lax.axis_index("c")
        base = wid * BPW
        pltpu.sync_copy(pos_u_h.at[pl.ds(base, BPW)], u_idx)
        pltpu.sync_copy(pos_v_h.at[pl.ds(base, BPW)], v_idx)
        pltpu.sync_copy(neg_h.at[pl.ds(base * NEG, BPW * NEG)], n_idx)

        sems = (sem0, sem1)
        NCHUNK = 128

        def fire(r, slot):
            cps = [
                pltpu.async_copy(
                    u_tab.at[u_idx.at[pl.ds(r * C, C)]],
                    u_rows.at[slot], sems[slot]),
                pltpu.async_copy(
                    v_tab.at[v_idx.at[pl.ds(r * C, C)]],
                    v_rows.at[slot], sems[slot]),
            ]
            for s in range(0, C * NEG, NCHUNK):
                n = min(NCHUNK, C * NEG - s)
                cps.append(pltpu.async_copy(
                    v_tab.at[n_idx.at[pl.ds(r * C * NEG + s, n)]],
                    n_rows.at[slot].at[pl.ds(s, n)], sems[slot]))
            return tuple(cps)

        UNROLL = 4
        iota = lax.iota(jnp.int32, L)
        pending = fire(0, 0)
        for r in range(ROUNDS):
            slot = r % 2
            for cpy in pending:
                cpy.wait()
            if r + 1 < ROUNDS:
                pending = fire(r + 1, 1 - slot)
            ur = u_rows.at[slot]
            vr = v_rows.at[slot]
            nr = n_rows.at[slot]

            def gbody(g, _, ur=ur, vr=vr, nr=nr, r=r):
                e = iota + g * L
                e10 = e * NEG
                col = r * C + g * L

                def dbody(j, accs, e=e, e10=e10, ur=ur, vr=vr, nr=nr):
                    accs = list(accs)
                    for jj in range(UNROLL):
                        dv = jnp.bitwise_and(iota + (j * UNROLL + jj), D - 1)
                        u_d = plsc.load_gather(ur, [e, dv])
                        v_d = plsc.load_gather(vr, [e, dv])
                        accs[0] = accs[0] + u_d * v_d
                        for kk in range(NEG):
                            n_d = plsc.load_gather(nr, [e10 + kk, dv])
                            accs[1 + kk] = accs[1 + kk] - n_d * u_d
                    return tuple(accs)

                accs = lax.fori_loop(
                    0, D // UNROLL, dbody,
                    tuple(jnp.zeros((L,), jnp.float32) for _ in range(KOUT)))
                for kk in range(KOUT):
                    stage[kk, pl.ds(col, L)] = accs[kk]
                return 0

            lax.fori_loop(0, C // L, gbody, 0)

        pltpu.sync_copy(stage, out_h.at[wid])

    return k(pos_u, pos_v, neg_flat, u_table, v_table)


def _tc_loss(scores2d):
    def body(s_ref, o_ref):
        x = s_ref[...]
        ls = jnp.where(x < 0.0, x, 0.0) - jnp.log1p(jnp.exp(-jnp.abs(x)))
        o_ref[0, 0] = -jnp.sum(ls)

    return pl.pallas_call(
        body,
        out_shape=jax.ShapeDtypeStruct((1, 1), jnp.float32),
        out_specs=pl.BlockSpec(memory_space=pltpu.SMEM),
    )(scores2d)


@jax.jit
def kernel(pos_u, pos_v, neg_v, u_table, v_table):
    neg_flat = neg_v.astype(jnp.int32).reshape(-1)
    scores = _sc_scores(pos_u.astype(jnp.int32), pos_v.astype(jnp.int32),
                        neg_flat, u_table, v_table)
    loss = _tc_loss(scores.reshape(NW * KOUT, BPW))
    return loss[0, 0]


# speedup vs baseline: 4.8227x; 1.6033x over previous
import functools

import jax
import jax.numpy as jnp
from jax import lax
from jax.experimental import pallas as pl
from jax.experimental.pallas import tpu as pltpu
from jax.experimental.pallas import tpu_sc as plsc

VOCAB = 1000000
D = 64
B = 16384
NEG = 10

NC, NS, L = 2, 16, 16
NW = NC * NS
BPW = B // NW
C = 32
ROUNDS = BPW // C
KOUT = 1 + NEG
PD = 2 * D
NSLAB = VOCAB // 128
TAIL = VOCAB - NSLAB * 128
SPT = NSLAB // NW

_sc_params = pltpu.CompilerParams(
    needs_layout_passes=False, use_tc_tiling_on_sc=True)


def _mesh():
    return plsc.VectorSubcoreMesh(
        core_axis_name="c", subcore_axis_name="s",
        num_cores=NC, num_subcores=NS)


def _sc_transpose(ut_t, vt_t):
    @functools.partial(
        pl.kernel,
        mesh=_mesh(),
        compiler_params=_sc_params,
        out_type=[
            jax.ShapeDtypeStruct((VOCAB // 2, PD), jnp.float32),
            jax.ShapeDtypeStruct((VOCAB // 2, PD), jnp.float32),
        ],
        scratch_types=[
            pltpu.VMEM((2, D, 128), jnp.float32),
            pltpu.VMEM((2, D, 128), jnp.float32),
            pltpu.VMEM((D, TAIL), jnp.float32),
            pltpu.SemaphoreType.DMA,
            pltpu.SemaphoreType.DMA,
            pltpu.SemaphoreType.DMA,
            pltpu.SemaphoreType.DMA,
        ],
    )
    def kt(ut_h, vt_h, u2_h, v2_h, in_s, out_s, tin, is0, is1, os0, os1):
        wid = lax.axis_index("s") * NC + lax.axis_index("c")
        iota = lax.iota(jnp.int32, L)
        isems = (is0, is1)
        osems = (os0, os1)

        def transpose_buf(src, dst, ngroups=64 // L):
            def tbody(t, _):
                dv = jnp.bitwise_and(iota + t, D - 1)
                for g in range(ngroups):
                    j16 = iota + g * L
                    j2 = j16 * 2
                    ev = plsc.load_gather(src, [dv, j2])
                    ov = plsc.load_gather(src, [dv, j2 + 1])
                    plsc.store_scatter(dst, [j16, dv], ev)
                    plsc.store_scatter(dst, [j16, dv + D], ov)
                return 0

            lax.fori_loop(0, D, tbody, 0)

        def run_table(tab_h, out_h):
            def col0(si):
                return pl.multiple_of((wid + NW * si) * 128, 128)

            def row0(si):
                return pl.multiple_of((wid + NW * si) * D, 8)

            def fire_in(si, b):
                pltpu.async_copy(
                    tab_h.at[:, pl.ds(col0(si), 128)], in_s.at[b], isems[b])

            def wait_in(b):
                pltpu.make_async_copy(
                    tab_h.at[:, pl.ds(0, 128)], in_s.at[b], isems[b]).wait()

            def fire_out(si, b):
                pltpu.async_copy(
                    out_s.at[b], out_h.at[pl.ds(row0(si), D)], osems[b])

            def wait_out(b):
                pltpu.make_async_copy(
                    out_s.at[b], out_h.at[pl.ds(0, D)], osems[b]).wait()

            fire_in(0, 0)
            fire_in(1, 1)

            def body2(si0, _):
                for bb in range(2):
                    si = si0 * 2 + bb
                    wait_in(bb)

                    @pl.when(si >= 2)
                    def _():
                        wait_out(bb)

                    transpose_buf(in_s.at[bb], out_s.at[bb])
                    fire_out(si, bb)

                    @pl.when(si + 2 < SPT)
                    def _():
                        fire_in(si + 2, bb)
                return 0

            lax.fori_loop(0, SPT // 2, body2, 0)
            wait_out(0)
            wait_out(1)

        run_table(ut_h, u2_h)
        run_table(vt_h, v2_h)

        @pl.when(wid == 7)
        def _():
            for tab_h, out_h in ((ut_h, u2_h), (vt_h, v2_h)):
                pltpu.sync_copy(
                    tab_h.at[:, pl.ds(NSLAB * 128, TAIL)], tin)
                transpose_buf(tin, out_s.at[0], ngroups=TAIL // 2 // L)
                pltpu.sync_copy(
                    out_s.at[0].at[pl.ds(0, TAIL // 2)],
                    out_h.at[pl.ds(NSLAB * D, TAIL // 2)])

    return kt(ut_t, vt_t)


def _sc_scores(pos_u, pos_v, neg_flat, u2, v2):
    @functools.partial(
        pl.kernel,
        mesh=_mesh(),
        compiler_params=_sc_params,
        out_type=jax.ShapeDtypeStruct((NW, KOUT, BPW), jnp.float32),
        scratch_types=[
            pltpu.VMEM((BPW,), jnp.int32),
            pltpu.VMEM((BPW,), jnp.int32),
            pltpu.VMEM((BPW * NEG,), jnp.int32),
            pltpu.VMEM((BPW,), jnp.int32),
            pltpu.VMEM((BPW,), jnp.int32),
            pltpu.VMEM((BPW * NEG,), jnp.int32),
            pltpu.VMEM((2, C, PD), jnp.float32),
            pltpu.VMEM((2, C, PD), jnp.float32),
            pltpu.VMEM((2, C * NEG, PD), jnp.float32),
            pltpu.VMEM((KOUT, BPW), jnp.float32),
            pltpu.SemaphoreType.DMA,
            pltpu.SemaphoreType.DMA,
        ],
    )
    def k(pos_u_h, pos_v_h, neg_h, u_tab, v_tab, out_h,
          u_idx, v_idx, n_idx, u_phys, v_phys, n_phys,
          u_rows, v_rows, n_rows, stage, sem0, sem1):
        wid = lax.axis_index("s") * NC + lax.axis_index("c")
        base = wid * BPW
        pltpu.sync_copy(pos_u_h.at[pl.ds(base, BPW)], u_idx)
        pltpu.sync_copy(pos_v_h.at[pl.ds(base, BPW)], v_idx)
        pltpu.sync_copy(neg_h.at[pl.ds(base * NEG, BPW * NEG)], n_idx)

        def make_halver(src, dst):
            def halver(i, acc):
                dst[pl.ds(i * L, L)] = lax.shift_right_logical(
                    src[pl.ds(i * L, L)], 1)
                return acc
            return halver

        lax.fori_loop(0, BPW // L, make_halver(u_idx, u_phys), 0)
        lax.fori_loop(0, BPW // L, make_halver(v_idx, v_phys), 0)
        lax.fori_loop(0, BPW * NEG // L, make_halver(n_idx, n_phys), 0)

        sems = (sem0, sem1)
        NCHUNK = 128

        def fire(r, slot):
            cps = [
                pltpu.async_copy(
                    u_tab.at[u_phys.at[pl.ds(r * C, C)]], u_rows.at[slot],
                    sems[slot]),
                pltpu.async_copy(
                    v_tab.at[v_phys.at[pl.ds(r * C, C)]], v_rows.at[slot],
                    sems[slot]),
            ]
            for s in range(0, C * NEG, NCHUNK):
                n = min(NCHUNK, C * NEG - s)
                cps.append(pltpu.async_copy(
                    v_tab.at[n_phys.at[pl.ds(r * C * NEG + s, n)]],
                    n_rows.at[slot].at[pl.ds(s, n)], sems[slot]))
            return tuple(cps)

        UNROLL = 4
        iota = lax.iota(jnp.int32, L)
        pending = fire(0, 0)
        for r in range(ROUNDS):
            slot = r % 2
            for cpy in pending:
                cpy.wait()
            if r + 1 < ROUNDS:
                pending = fire(r + 1, 1 - slot)
            ur = u_rows.at[slot]
            vr = v_rows.at[slot]
            nr = n_rows.at[slot]

            def gbody(g, _, ur=ur, vr=vr, nr=nr, r=r):
                e = iota + g * L
                e10 = e * NEG
                col = r * C + g * L
                uo = lax.shift_left(
                    jnp.bitwise_and(u_idx[pl.ds(col, L)], 1), 6)
                vo = lax.shift_left(
                    jnp.bitwise_and(v_idx[pl.ds(col, L)], 1), 6)
                gpos10 = (iota + col) * NEG
                nos = [
                    lax.shift_left(
                        jnp.bitwise_and(
                            plsc.load_gather(n_idx, [gpos10 + kk]), 1), 6)
                    for kk in range(NEG)
                ]

                def dbody(j, accs, e=e, e10=e10, uo=uo, vo=vo, nos=nos,
                          ur=ur, vr=vr, nr=nr):
                    accs = list(accs)
                    for jj in range(UNROLL):
                        dv = jnp.bitwise_and(iota + (j * UNROLL + jj), D - 1)
                        u_d = plsc.load_gather(ur, [e, uo + dv])
                        v_d = plsc.load_gather(vr, [e, vo + dv])
                        accs[0] = accs[0] + u_d * v_d
                        for kk in range(NEG):
                            n_d = plsc.load_gather(nr, [e10 + kk, nos[kk] + dv])
                            accs[1 + kk] = accs[1 + kk] - n_d * u_d
                    return tuple(accs)

                accs = lax.fori_loop(
                    0, D // UNROLL, dbody,
                    tuple(jnp.zeros((L,), jnp.float32) for _ in range(KOUT)))
                for kk in range(KOUT):
                    stage[kk, pl.ds(col, L)] = accs[kk]
                return 0

            lax.fori_loop(0, C // L, gbody, 0)

        pltpu.sync_copy(stage, out_h.at[wid])

    return k(pos_u, pos_v, neg_flat, u2, v2)


def _tc_loss(scores2d):
    def body(s_ref, o_ref):
        x = s_ref[...]
        ls = jnp.where(x < 0.0, x, 0.0) - jnp.log1p(jnp.exp(-jnp.abs(x)))
        o_ref[0, 0] = -jnp.sum(ls)

    return pl.pallas_call(
        body,
        out_shape=jax.ShapeDtypeStruct((1, 1), jnp.float32),
        out_specs=pl.BlockSpec(memory_space=pltpu.SMEM),
    )(scores2d)


@jax.jit
def kernel(pos_u, pos_v, neg_v, u_table, v_table):
    neg_flat = neg_v.astype(jnp.int32).reshape(-1)
    u2, v2 = _sc_transpose(u_table.T, v_table.T)
    scores = _sc_scores(pos_u.astype(jnp.int32), pos_v.astype(jnp.int32),
                        neg_flat, u2, v2)
    loss = _tc_loss(scores.reshape(NW * KOUT, BPW))
    return loss[0, 0]
